# Initial kernel scaffold; baseline (speedup 1.0000x reference)
import jax
import jax.numpy as jnp
from jax.experimental import pallas as pl


def kernel(features, edge_index, W1, b1, W2, b2):
    raise NotImplementedError("write your pallas kernel here")


# speedup vs baseline: 2.7521x; 2.7521x over previous
import functools

import jax
import jax.numpy as jnp
from jax import lax
from jax.experimental import pallas as pl
from jax.experimental.pallas import tpu as pltpu
from jax.experimental.pallas import tpu_sc as plsc

_NC = 2
_NS = 16
_CHUNK = 128


def _make_agg(N_pad, D, E_pad):
  NW = _NC * _NS
  EPW = E_pad // NW
  n_chunks = EPW // _CHUNK
  rows_per_tile = N_pad // _NS
  mesh = plsc.VectorSubcoreMesh(core_axis_name="c", subcore_axis_name="s")

  @functools.partial(
      pl.kernel,
      mesh=mesh,
      out_type=jax.ShapeDtypeStruct((_NC * N_pad, D), jnp.float32),
      scratch_types=[
          pltpu.VMEM((n_chunks, _CHUNK), jnp.int32),
          pltpu.VMEM((n_chunks, _CHUNK), jnp.int32),
          pltpu.VMEM((_CHUNK, D), jnp.float32),
          pltpu.VMEM_SHARED((N_pad, D), jnp.float32),
          pltpu.SemaphoreType.DMA,
      ],
  )
  def agg(table_hbm, src_hbm, dst_hbm, zeros_hbm, out_hbm,
          src_v, dst_v, rows_v, acc_sh, sem):
    cid = lax.axis_index("c")
    sid = lax.axis_index("s")
    wid = sid * _NC + cid
    row0 = sid * rows_per_tile

    @pl.when(cid == 0)
    def _():
      pltpu.sync_copy(table_hbm.at[pl.ds(row0, rows_per_tile)],
                      acc_sh.at[pl.ds(row0, rows_per_tile)])

    @pl.when(cid != 0)
    def _():
      pltpu.sync_copy(zeros_hbm, acc_sh.at[pl.ds(row0, rows_per_tile)])

    chunk0 = wid * n_chunks
    pltpu.sync_copy(src_hbm.at[pl.ds(chunk0, n_chunks)], src_v)
    pltpu.sync_copy(dst_hbm.at[pl.ds(chunk0, n_chunks)], dst_v)
    plsc.subcore_barrier()

    def body(j, carry):
      pltpu.async_copy(table_hbm.at[src_v.at[j]], rows_v, sem).wait()
      pltpu.sync_copy(rows_v, acc_sh.at[dst_v.at[j]], add=True)
      return carry

    lax.fori_loop(0, n_chunks, body, 0)
    plsc.subcore_barrier()
    pltpu.sync_copy(acc_sh.at[pl.ds(row0, rows_per_tile)],
                    out_hbm.at[pl.ds(cid * N_pad + row0, rows_per_tile)])

  return agg


def _dense(p, w_t, b):
  N2, D = p.shape
  Np = N2 // 2
  H = w_t.shape[1]
  BM = 256
  grid = Np // BM

  def body(p0_ref, p1_ref, w_ref, b_ref, o_ref):
    h = p0_ref[...] + p1_ref[...]
    acc = jnp.dot(h, w_ref[...], preferred_element_type=jnp.float32)
    acc = acc + b_ref[...]
    o_ref[...] = jnp.where(acc > 0, acc, jnp.exp(acc) - 1.0)

  return pl.pallas_call(
      body,
      grid=(grid,),
      in_specs=[
          pl.BlockSpec((BM, D), lambda i: (i, 0)),
          pl.BlockSpec((BM, D), lambda i: (i + grid, 0)),
          pl.BlockSpec((D, H), lambda i: (0, 0)),
          pl.BlockSpec((1, H), lambda i: (0, 0)),
      ],
      out_specs=pl.BlockSpec((BM, H), lambda i: (i, 0)),
      out_shape=jax.ShapeDtypeStruct((Np, H), jnp.float32),
  )(p, p, w_t, b.reshape(1, H))


def kernel(features, edge_index, W1, b1, W2, b2):
  N, D = features.shape
  E = edge_index.shape[1]
  H = W1.shape[0]

  N_pad = ((N + 1 + 255) // 256) * 256
  step = _NC * _NS * _CHUNK * 8
  E_pad = ((E + step - 1) // step) * step

  feats_pad = jnp.pad(features, ((0, N_pad - N), (0, 0)))
  pad_e = E_pad - E
  src_p = jnp.concatenate(
      [edge_index[0], jnp.zeros((pad_e,), jnp.int32)]).reshape(-1, _CHUNK)
  dst_p = jnp.concatenate(
      [edge_index[1], jnp.full((pad_e,), N, jnp.int32)]).reshape(-1, _CHUNK)
  zeros_rows = jnp.zeros((N_pad // _NS, D), jnp.float32)

  agg1 = _make_agg(N_pad, D, E_pad)
  p1 = agg1(feats_pad, src_p, dst_p, zeros_rows)
  x = _dense(p1, W1.T, b1)

  agg2 = _make_agg(N_pad, H, E_pad)
  p2 = agg2(x, src_p, dst_p, zeros_rows)
  out = _dense(p2, W2.T, b2)
  return out[:N]


# speedup vs baseline: 8.5619x; 3.1111x over previous
import functools

import jax
import jax.numpy as jnp
from jax import lax
from jax.experimental import pallas as pl
from jax.experimental.pallas import tpu as pltpu
from jax.experimental.pallas import tpu_sc as plsc

_NC = 2
_NS = 16
_CHUNK = 128
_G = 8


def _make_agg(N_pad, D, E_pad):
  NW = _NC * _NS
  EPW = E_pad // NW
  n_chunks = EPW // _CHUNK
  n_groups = n_chunks // _G
  rows_per_tile = N_pad // _NS
  zchunks = rows_per_tile // _CHUNK
  mesh = plsc.VectorSubcoreMesh(core_axis_name="c", subcore_axis_name="s")

  @functools.partial(
      pl.kernel,
      mesh=mesh,
      out_type=jax.ShapeDtypeStruct((_NC * N_pad, D), jnp.float32),
      scratch_types=[
          pltpu.VMEM((2, _G, _CHUNK), jnp.int32),
          pltpu.VMEM((2, _G, _CHUNK), jnp.int32),
          pltpu.VMEM((2, _CHUNK, D), jnp.float32),
          pltpu.VMEM_SHARED((N_pad, D), jnp.float32),
          pltpu.SemaphoreType.DMA,
          pltpu.SemaphoreType.DMA,
          pltpu.SemaphoreType.DMA,
          pltpu.SemaphoreType.DMA,
          pltpu.SemaphoreType.DMA,
      ],
  )
  def agg(table_hbm, src_hbm, dst_hbm, zeros_hbm, out_hbm,
          src_v, dst_v, rows_v, acc_sh, g0, g1, s0, s1, si):
    sem_g = (g0, g1)
    sem_s = (s0, s1)
    cid = lax.axis_index("c")
    sid = lax.axis_index("s")
    wid = sid * _NC + cid
    row0 = sid * rows_per_tile
    chunk0 = wid * n_chunks

    pltpu.sync_copy(src_hbm.at[pl.ds(chunk0, _G)], src_v.at[0])
    pltpu.sync_copy(dst_hbm.at[pl.ds(chunk0, _G)], dst_v.at[0])

    pltpu.async_copy(table_hbm.at[src_v.at[0, 0]], rows_v.at[0], sem_g[0])

    pltpu.sync_copy(zeros_hbm, rows_v.at[1])
    for z in range(zchunks):
      pltpu.sync_copy(rows_v.at[1],
                      acc_sh.at[pl.ds(row0 + z * _CHUNK, _CHUNK)])
    plsc.subcore_barrier()

    def group_body(g, carry):
      gslot = lax.rem(g, 2)
      nslot = 1 - gslot
      for k in range(_G):
        j = g * _G + k
        b = k % 2
        pltpu.make_async_copy(
            table_hbm.at[src_v.at[gslot, k]], rows_v.at[b], sem_g[b]).wait()
        pltpu.async_copy(
            rows_v.at[b], acc_sh.at[dst_v.at[gslot, k]], sem_s[b], add=True)
        if k == 0:
          @pl.when(g >= 1)
          def _():
            pltpu.make_async_copy(
                rows_v.at[1], acc_sh.at[dst_v.at[nslot, _G - 1]],
                sem_s[1]).wait()
          @pl.when(g + 1 < n_groups)
          def _():
            nxt = chunk0 + (g + 1) * _G
            pltpu.async_copy(src_hbm.at[pl.ds(nxt, _G)], src_v.at[nslot], si)
            pltpu.async_copy(dst_hbm.at[pl.ds(nxt, _G)], dst_v.at[nslot], si)
        else:
          pltpu.make_async_copy(
              rows_v.at[1 - b], acc_sh.at[dst_v.at[gslot, k - 1]],
              sem_s[1 - b]).wait()
        if k < _G - 1:
          pltpu.async_copy(
              table_hbm.at[src_v.at[gslot, k + 1]], rows_v.at[1 - b],
              sem_g[1 - b])
        else:
          @pl.when(g + 1 < n_groups)
          def _():
            nxt = chunk0 + (g + 1) * _G
            pltpu.make_async_copy(
                src_hbm.at[pl.ds(nxt, _G)], src_v.at[nslot], si).wait()
            pltpu.make_async_copy(
                dst_hbm.at[pl.ds(nxt, _G)], dst_v.at[nslot], si).wait()
            pltpu.async_copy(
                table_hbm.at[src_v.at[nslot, 0]], rows_v.at[1 - b],
                sem_g[1 - b])
      return carry

    lax.fori_loop(0, n_groups, group_body, 0)

    last = lax.rem(n_groups - 1, 2)
    pltpu.make_async_copy(
        rows_v.at[1], acc_sh.at[dst_v.at[last, _G - 1]], sem_s[1]).wait()
    plsc.subcore_barrier()

    for z in range(zchunks):
      r = row0 + z * _CHUNK
      buf = rows_v.at[z % 2]
      pltpu.sync_copy(acc_sh.at[pl.ds(r, _CHUNK)], buf)
      pltpu.sync_copy(buf, out_hbm.at[pl.ds(cid * N_pad + r, _CHUNK)])

  return agg


def _dense(base, p, w_t, b):
  Np, D = base.shape
  H = w_t.shape[1]
  BM = 256
  grid = Np // BM

  def body(base_ref, p0_ref, p1_ref, w_ref, b_ref, o_ref):
    h = base_ref[...] + p0_ref[...] + p1_ref[...]
    acc = jnp.dot(h, w_ref[...], preferred_element_type=jnp.float32)
    acc = acc + b_ref[...]
    o_ref[...] = jnp.where(acc > 0, acc, jnp.exp(acc) - 1.0)

  return pl.pallas_call(
      body,
      grid=(grid,),
      in_specs=[
          pl.BlockSpec((BM, D), lambda i: (i, 0)),
          pl.BlockSpec((BM, D), lambda i: (i, 0)),
          pl.BlockSpec((BM, D), lambda i: (i + grid, 0)),
          pl.BlockSpec((D, H), lambda i: (0, 0)),
          pl.BlockSpec((1, H), lambda i: (0, 0)),
      ],
      out_specs=pl.BlockSpec((BM, H), lambda i: (i, 0)),
      out_shape=jax.ShapeDtypeStruct((Np, H), jnp.float32),
  )(base, p, p, w_t, b.reshape(1, H))


def kernel(features, edge_index, W1, b1, W2, b2):
  N, D = features.shape
  E = edge_index.shape[1]
  H = W1.shape[0]

  N_pad = ((N + 8 + 255) // 256) * 256
  step = _NC * _NS * _CHUNK * _G
  E_pad = ((E + step - 1) // step) * step

  feats_pad = jnp.pad(features, ((0, N_pad - N), (0, 0)))
  pad_e = E_pad - E
  cyc = jnp.arange(pad_e, dtype=jnp.int32) % 8
  src_p = jnp.concatenate([edge_index[0], cyc]).reshape(-1, _CHUNK)
  dst_p = jnp.concatenate([edge_index[1], N + cyc]).reshape(-1, _CHUNK)
  zeros_rows = jnp.zeros((_CHUNK, D), jnp.float32)

  agg1 = _make_agg(N_pad, D, E_pad)
  p1 = agg1(feats_pad, src_p, dst_p, zeros_rows)
  x = _dense(feats_pad, p1, W1.T, b1)

  agg2 = _make_agg(N_pad, H, E_pad)
  p2 = agg2(x, src_p, dst_p, zeros_rows)
  out = _dense(x, p2, W2.T, b2)
  return out[:N]


# speedup vs baseline: 9.9277x; 1.1595x over previous
import functools

import jax
import jax.numpy as jnp
from jax import lax
from jax.experimental import pallas as pl
from jax.experimental.pallas import tpu as pltpu
from jax.experimental.pallas import tpu_sc as plsc

_NC = 2
_NS = 16
_CHUNK = 128
_G = 8


def _make_agg(N_pad, D, E_pad):
  NW = _NC * _NS
  EPW = E_pad // NW
  n_chunks = EPW // _CHUNK
  n_groups = n_chunks // _G
  rows_per_tile = N_pad // _NS
  zchunks = rows_per_tile // _CHUNK
  mesh = plsc.VectorSubcoreMesh(core_axis_name="c", subcore_axis_name="s")

  @functools.partial(
      pl.kernel,
      mesh=mesh,
      out_type=jax.ShapeDtypeStruct((_NC * N_pad, D), jnp.float32),
      scratch_types=[
          pltpu.VMEM((2, _G, _CHUNK), jnp.int32),
          pltpu.VMEM((2, _G, _CHUNK), jnp.int32),
          pltpu.VMEM((2, _CHUNK, D), jnp.float32),
          pltpu.VMEM_SHARED((N_pad, D), jnp.float32),
          pltpu.SemaphoreType.DMA,
          pltpu.SemaphoreType.DMA,
          pltpu.SemaphoreType.DMA,
          pltpu.SemaphoreType.DMA,
          pltpu.SemaphoreType.DMA,
      ],
  )
  def agg(table_hbm, src_hbm, dst_hbm, zeros_hbm, out_hbm,
          src_v, dst_v, rows_v, acc_sh, g0, g1, s0, s1, si):
    sem_g = (g0, g1)
    sem_s = (s0, s1)
    cid = lax.axis_index("c")
    sid = lax.axis_index("s")
    wid = sid * _NC + cid
    row0 = sid * rows_per_tile
    chunk0 = wid * n_chunks

    pltpu.sync_copy(src_hbm.at[pl.ds(chunk0, _G)], src_v.at[0])
    pltpu.sync_copy(dst_hbm.at[pl.ds(chunk0, _G)], dst_v.at[0])

    pltpu.async_copy(table_hbm.at[src_v.at[0, 0]], rows_v.at[0], sem_g[0])

    pltpu.sync_copy(zeros_hbm, rows_v.at[1])
    for z in range(zchunks):
      pltpu.sync_copy(rows_v.at[1],
                      acc_sh.at[pl.ds(row0 + z * _CHUNK, _CHUNK)])
    plsc.subcore_barrier()

    def group_body(g, carry):
      gslot = lax.rem(g, 2)
      nslot = 1 - gslot
      for k in range(_G):
        j = g * _G + k
        b = k % 2
        if k == 0:
          @pl.when(g >= 1)
          def _():
            pltpu.make_async_copy(
                rows_v.at[1], acc_sh.at[dst_v.at[nslot, _G - 1]],
                sem_s[1]).wait()
          @pl.when(g + 1 < n_groups)
          def _():
            nxt = chunk0 + (g + 1) * _G
            pltpu.async_copy(src_hbm.at[pl.ds(nxt, _G)], src_v.at[nslot], si)
            pltpu.async_copy(dst_hbm.at[pl.ds(nxt, _G)], dst_v.at[nslot], si)
        else:
          pltpu.make_async_copy(
              rows_v.at[1 - b], acc_sh.at[dst_v.at[gslot, k - 1]],
              sem_s[1 - b]).wait()
        if k < _G - 1:
          pltpu.async_copy(
              table_hbm.at[src_v.at[gslot, k + 1]], rows_v.at[1 - b],
              sem_g[1 - b])
        else:
          @pl.when(g + 1 < n_groups)
          def _():
            nxt = chunk0 + (g + 1) * _G
            pltpu.make_async_copy(
                src_hbm.at[pl.ds(nxt, _G)], src_v.at[nslot], si).wait()
            pltpu.make_async_copy(
                dst_hbm.at[pl.ds(nxt, _G)], dst_v.at[nslot], si).wait()
            pltpu.async_copy(
                table_hbm.at[src_v.at[nslot, 0]], rows_v.at[1 - b],
                sem_g[1 - b])
        pltpu.make_async_copy(
            table_hbm.at[src_v.at[gslot, k]], rows_v.at[b], sem_g[b]).wait()
        pltpu.async_copy(
            rows_v.at[b], acc_sh.at[dst_v.at[gslot, k]], sem_s[b], add=True)
      return carry

    lax.fori_loop(0, n_groups, group_body, 0)

    last = lax.rem(n_groups - 1, 2)
    pltpu.make_async_copy(
        rows_v.at[1], acc_sh.at[dst_v.at[last, _G - 1]], sem_s[1]).wait()
    plsc.subcore_barrier()

    for z in range(zchunks):
      r = row0 + z * _CHUNK
      buf = rows_v.at[z % 2]
      pltpu.sync_copy(acc_sh.at[pl.ds(r, _CHUNK)], buf)
      pltpu.sync_copy(buf, out_hbm.at[pl.ds(cid * N_pad + r, _CHUNK)])

  return agg


def _dense(base, p, w_t, b):
  Np, D = base.shape
  H = w_t.shape[1]
  BM = 256
  grid = Np // BM

  def body(base_ref, p0_ref, p1_ref, w_ref, b_ref, o_ref):
    h = base_ref[...] + p0_ref[...] + p1_ref[...]
    acc = jnp.dot(h, w_ref[...], preferred_element_type=jnp.float32)
    acc = acc + b_ref[...]
    o_ref[...] = jnp.where(acc > 0, acc, jnp.exp(acc) - 1.0)

  return pl.pallas_call(
      body,
      grid=(grid,),
      in_specs=[
          pl.BlockSpec((BM, D), lambda i: (i, 0)),
          pl.BlockSpec((BM, D), lambda i: (i, 0)),
          pl.BlockSpec((BM, D), lambda i: (i + grid, 0)),
          pl.BlockSpec((D, H), lambda i: (0, 0)),
          pl.BlockSpec((1, H), lambda i: (0, 0)),
      ],
      out_specs=pl.BlockSpec((BM, H), lambda i: (i, 0)),
      out_shape=jax.ShapeDtypeStruct((Np, H), jnp.float32),
  )(base, p, p, w_t, b.reshape(1, H))


def kernel(features, edge_index, W1, b1, W2, b2):
  N, D = features.shape
  E = edge_index.shape[1]
  H = W1.shape[0]

  N_pad = ((N + 8 + 255) // 256) * 256
  step = _NC * _NS * _CHUNK * _G
  E_pad = ((E + step - 1) // step) * step

  feats_pad = jnp.pad(features, ((0, N_pad - N), (0, 0)))
  pad_e = E_pad - E
  cyc = jnp.arange(pad_e, dtype=jnp.int32) % 8
  src_p = jnp.concatenate([edge_index[0], cyc]).reshape(-1, _CHUNK)
  dst_p = jnp.concatenate([edge_index[1], N + cyc]).reshape(-1, _CHUNK)
  zeros_rows = jnp.zeros((_CHUNK, D), jnp.float32)

  agg1 = _make_agg(N_pad, D, E_pad)
  p1 = agg1(feats_pad, src_p, dst_p, zeros_rows)
  x = _dense(feats_pad, p1, W1.T, b1)

  agg2 = _make_agg(N_pad, H, E_pad)
  p2 = agg2(x, src_p, dst_p, zeros_rows)
  out = _dense(x, p2, W2.T, b2)
  return out[:N]


# speedup vs baseline: 11.7351x; 1.1821x over previous
import functools

import jax
import jax.numpy as jnp
from jax import lax
from jax.experimental import pallas as pl
from jax.experimental.pallas import tpu as pltpu
from jax.experimental.pallas import tpu_sc as plsc

_NC = 2
_NS = 16
_CHUNK = 128
_G = 8


def _make_agg(N_table, N_pad, D, E_pad):
  NW = _NC * _NS
  EPW = E_pad // NW
  n_chunks = EPW // _CHUNK
  n_groups = n_chunks // _G
  rows_per_tile = N_pad // _NS
  zchunks = rows_per_tile // _CHUNK
  mesh = plsc.VectorSubcoreMesh(core_axis_name="c", subcore_axis_name="s")
  out_sds = jax.ShapeDtypeStruct((N_pad, D), jnp.float32)

  @functools.partial(
      pl.kernel,
      mesh=mesh,
      out_type=(out_sds, out_sds),
      scratch_types=[
          pltpu.VMEM((2, _G, _CHUNK), jnp.int32),
          pltpu.VMEM((2, _G, _CHUNK), jnp.int32),
          pltpu.VMEM((2, _CHUNK, D), jnp.float32),
          pltpu.VMEM_SHARED((N_pad, D), jnp.float32),
          pltpu.SemaphoreType.DMA,
          pltpu.SemaphoreType.DMA,
          pltpu.SemaphoreType.DMA,
          pltpu.SemaphoreType.DMA,
          pltpu.SemaphoreType.DMA,
      ],
  )
  def agg(table_hbm, src_hbm, dst_hbm, zeros_hbm, out0_hbm, out1_hbm,
          src_v, dst_v, rows_v, acc_sh, g0, g1, s0, s1, si):
    sem_g = (g0, g1)
    sem_s = (s0, s1)
    cid = lax.axis_index("c")
    sid = lax.axis_index("s")
    wid = sid * _NC + cid
    row0 = sid * rows_per_tile
    chunk0 = wid * n_chunks

    pltpu.sync_copy(src_hbm.at[pl.ds(chunk0, _G)], src_v.at[0])
    pltpu.sync_copy(dst_hbm.at[pl.ds(chunk0, _G)], dst_v.at[0])

    pltpu.async_copy(table_hbm.at[src_v.at[0, 0]], rows_v.at[0], sem_g[0])

    pltpu.sync_copy(zeros_hbm, rows_v.at[1])
    for z in range(zchunks):
      pltpu.sync_copy(rows_v.at[1],
                      acc_sh.at[pl.ds(row0 + z * _CHUNK, _CHUNK)])
    plsc.subcore_barrier()

    def group_body(g, carry):
      gslot = lax.rem(g, 2)
      nslot = 1 - gslot
      for k in range(_G):
        b = k % 2
        if k == 0:
          @pl.when(g >= 1)
          def _():
            pltpu.make_async_copy(
                rows_v.at[1], acc_sh.at[dst_v.at[nslot, _G - 1]],
                sem_s[1]).wait()
          @pl.when(g + 1 < n_groups)
          def _():
            nxt = chunk0 + (g + 1) * _G
            pltpu.async_copy(src_hbm.at[pl.ds(nxt, _G)], src_v.at[nslot], si)
            pltpu.async_copy(dst_hbm.at[pl.ds(nxt, _G)], dst_v.at[nslot], si)
        else:
          pltpu.make_async_copy(
              rows_v.at[1 - b], acc_sh.at[dst_v.at[gslot, k - 1]],
              sem_s[1 - b]).wait()
        if k < _G - 1:
          pltpu.async_copy(
              table_hbm.at[src_v.at[gslot, k + 1]], rows_v.at[1 - b],
              sem_g[1 - b])
        else:
          @pl.when(g + 1 < n_groups)
          def _():
            nxt = chunk0 + (g + 1) * _G
            pltpu.make_async_copy(
                src_hbm.at[pl.ds(nxt, _G)], src_v.at[nslot], si).wait()
            pltpu.make_async_copy(
                dst_hbm.at[pl.ds(nxt, _G)], dst_v.at[nslot], si).wait()
            pltpu.async_copy(
                table_hbm.at[src_v.at[nslot, 0]], rows_v.at[1 - b],
                sem_g[1 - b])
        pltpu.make_async_copy(
            table_hbm.at[src_v.at[gslot, k]], rows_v.at[b], sem_g[b]).wait()
        pltpu.async_copy(
            rows_v.at[b], acc_sh.at[dst_v.at[gslot, k]], sem_s[b], add=True)
      return carry

    lax.fori_loop(0, n_groups, group_body, 0)

    last = lax.rem(n_groups - 1, 2)
    pltpu.make_async_copy(
        rows_v.at[1], acc_sh.at[dst_v.at[last, _G - 1]], sem_s[1]).wait()
    plsc.subcore_barrier()

    for z in range(zchunks):
      r = row0 + z * _CHUNK
      buf = rows_v.at[z % 2]
      pltpu.sync_copy(acc_sh.at[pl.ds(r, _CHUNK)], buf)

      @pl.when(cid == 0)
      def _():
        pltpu.sync_copy(buf, out0_hbm.at[pl.ds(r, _CHUNK)])

      @pl.when(cid != 0)
      def _():
        pltpu.sync_copy(buf, out1_hbm.at[pl.ds(r, _CHUNK)])

  return agg


def _dense(base, p0, p1, w_t, b, n_out):
  D = base.shape[1]
  H = w_t.shape[1]
  BM = 400
  grid = n_out // BM

  def body(base_ref, p0_ref, p1_ref, w_ref, b_ref, o_ref):
    h = base_ref[...] + p0_ref[...] + p1_ref[...]
    acc = jnp.dot(h, w_ref[...], preferred_element_type=jnp.float32)
    acc = acc + b_ref[...]
    o_ref[...] = jnp.where(acc > 0, acc, jnp.exp(acc) - 1.0)

  return pl.pallas_call(
      body,
      grid=(grid,),
      in_specs=[
          pl.BlockSpec((BM, D), lambda i: (i, 0)),
          pl.BlockSpec((BM, D), lambda i: (i, 0)),
          pl.BlockSpec((BM, D), lambda i: (i, 0)),
          pl.BlockSpec((D, H), lambda i: (0, 0)),
          pl.BlockSpec((1, H), lambda i: (0, 0)),
      ],
      out_specs=pl.BlockSpec((BM, H), lambda i: (i, 0)),
      out_shape=jax.ShapeDtypeStruct((n_out, H), jnp.float32),
  )(base, p0, p1, w_t, b.reshape(1, H))


def kernel(features, edge_index, W1, b1, W2, b2):
  N, D = features.shape
  E = edge_index.shape[1]
  H = W1.shape[0]
  NW = _NC * _NS

  N_pad = ((N + 8 + 255) // 256) * 256
  step = NW * _CHUNK * _G
  E_pad = ((E + step - 1) // step) * step

  src, dst = edge_index[0], edge_index[1]
  pad_e = E_pad - E
  if E % NW == 0 and pad_e % NW == 0:
    ppt = pad_e // NW
    cyc = jnp.arange(ppt, dtype=jnp.int32) % (N_pad - N)
    pad_blk = jnp.broadcast_to(cyc, (NW, ppt))
    src_p = jnp.concatenate(
        [src.reshape(NW, E // NW), pad_blk], axis=1).reshape(-1, _CHUNK)
    dst_p = jnp.concatenate(
        [dst.reshape(NW, E // NW), N + pad_blk], axis=1).reshape(-1, _CHUNK)
  else:
    cyc = jnp.arange(pad_e, dtype=jnp.int32) % (N_pad - N)
    src_p = jnp.concatenate([src, cyc]).reshape(-1, _CHUNK)
    dst_p = jnp.concatenate([dst, N + cyc]).reshape(-1, _CHUNK)
  zeros_rows = jnp.zeros((_CHUNK, D), jnp.float32)

  agg1 = _make_agg(N, N_pad, D, E_pad)
  p0, p1 = agg1(features, src_p, dst_p, zeros_rows)
  x = _dense(features, p0, p1, W1.T, b1, N)

  agg2 = _make_agg(N, N_pad, H, E_pad)
  q0, q1 = agg2(x, src_p, dst_p, zeros_rows)
  return _dense(x, q0, q1, W2.T, b2, N)


# speedup vs baseline: 11.9807x; 1.0209x over previous
import functools

import jax
import jax.numpy as jnp
from jax import lax
from jax.experimental import pallas as pl
from jax.experimental.pallas import tpu as pltpu
from jax.experimental.pallas import tpu_sc as plsc

_NC = 2
_NS = 16
_CHUNK = 128
_G = 8


def _make_agg(N_table, N_pad, D, E_pad):
  NW = _NC * _NS
  EPW = E_pad // NW
  n_chunks = EPW // _CHUNK
  n_groups = n_chunks // _G
  rows_per_tile = N_pad // _NS
  zchunks = rows_per_tile // _CHUNK
  mesh = plsc.VectorSubcoreMesh(core_axis_name="c", subcore_axis_name="s")
  out_sds = jax.ShapeDtypeStruct((N_pad, D), jnp.float32)

  @functools.partial(
      pl.kernel,
      mesh=mesh,
      out_type=(out_sds, out_sds),
      scratch_types=[
          pltpu.VMEM((2, _G, _CHUNK), jnp.int32),
          pltpu.VMEM((2, _G, _CHUNK), jnp.int32),
          pltpu.VMEM((2, _CHUNK, D), jnp.float32),
          pltpu.VMEM_SHARED((N_pad, D), jnp.float32),
          pltpu.SemaphoreType.DMA,
          pltpu.SemaphoreType.DMA,
          pltpu.SemaphoreType.DMA,
          pltpu.SemaphoreType.DMA,
          pltpu.SemaphoreType.DMA,
      ],
  )
  def agg(table_hbm, src_hbm, dst_hbm, zeros_hbm, out0_hbm, out1_hbm,
          src_v, dst_v, rows_v, acc_sh, g0, g1, s0, s1, si):
    sem_g = (g0, g1)
    sem_s = (s0, s1)
    cid = lax.axis_index("c")
    sid = lax.axis_index("s")
    wid = sid * _NC + cid
    row0 = sid * rows_per_tile
    chunk0 = wid * n_chunks

    pltpu.sync_copy(src_hbm.at[pl.ds(chunk0, _G)], src_v.at[0])
    pltpu.async_copy(table_hbm.at[src_v.at[0, 0]], rows_v.at[0], sem_g[0])

    pltpu.sync_copy(zeros_hbm, rows_v.at[1])
    for z in range(zchunks):
      pltpu.async_copy(rows_v.at[1],
                       acc_sh.at[pl.ds(row0 + z * _CHUNK, _CHUNK)], s0)
    pltpu.sync_copy(dst_hbm.at[pl.ds(chunk0, _G)], dst_v.at[0])
    for z in range(zchunks):
      pltpu.make_async_copy(
          rows_v.at[1], acc_sh.at[pl.ds(row0 + z * _CHUNK, _CHUNK)],
          s0).wait()
    plsc.subcore_barrier()

    def group_body(g, carry):
      gslot = lax.rem(g, 2)
      nslot = 1 - gslot
      for k in range(_G):
        b = k % 2
        if k == 0:
          @pl.when(g >= 1)
          def _():
            pltpu.make_async_copy(
                rows_v.at[1], acc_sh.at[dst_v.at[nslot, _G - 1]],
                sem_s[1]).wait()
          @pl.when(g + 1 < n_groups)
          def _():
            nxt = chunk0 + (g + 1) * _G
            pltpu.async_copy(src_hbm.at[pl.ds(nxt, _G)], src_v.at[nslot], si)
            pltpu.async_copy(dst_hbm.at[pl.ds(nxt, _G)], dst_v.at[nslot], si)
        else:
          pltpu.make_async_copy(
              rows_v.at[1 - b], acc_sh.at[dst_v.at[gslot, k - 1]],
              sem_s[1 - b]).wait()
        if k < _G - 1:
          pltpu.async_copy(
              table_hbm.at[src_v.at[gslot, k + 1]], rows_v.at[1 - b],
              sem_g[1 - b])
        else:
          @pl.when(g + 1 < n_groups)
          def _():
            nxt = chunk0 + (g + 1) * _G
            pltpu.make_async_copy(
                src_hbm.at[pl.ds(nxt, _G)], src_v.at[nslot], si).wait()
            pltpu.make_async_copy(
                dst_hbm.at[pl.ds(nxt, _G)], dst_v.at[nslot], si).wait()
            pltpu.async_copy(
                table_hbm.at[src_v.at[nslot, 0]], rows_v.at[1 - b],
                sem_g[1 - b])
        pltpu.make_async_copy(
            table_hbm.at[src_v.at[gslot, k]], rows_v.at[b], sem_g[b]).wait()
        pltpu.async_copy(
            rows_v.at[b], acc_sh.at[dst_v.at[gslot, k]], sem_s[b], add=True)
      return carry

    lax.fori_loop(0, n_groups, group_body, 0)

    last = lax.rem(n_groups - 1, 2)
    pltpu.make_async_copy(
        rows_v.at[1], acc_sh.at[dst_v.at[last, _G - 1]], sem_s[1]).wait()
    plsc.subcore_barrier()

    def wb_in(z, b):
      pltpu.async_copy(acc_sh.at[pl.ds(row0 + z * _CHUNK, _CHUNK)],
                       rows_v.at[b], sem_g[b])

    def wb_out(z, b):
      r = row0 + z * _CHUNK

      @pl.when(cid == 0)
      def _():
        pltpu.async_copy(rows_v.at[b], out0_hbm.at[pl.ds(r, _CHUNK)],
                         sem_s[b])

      @pl.when(cid != 0)
      def _():
        pltpu.async_copy(rows_v.at[b], out1_hbm.at[pl.ds(r, _CHUNK)],
                         sem_s[b])

    def wb_wait_in(z, b):
      pltpu.make_async_copy(acc_sh.at[pl.ds(row0 + z * _CHUNK, _CHUNK)],
                            rows_v.at[b], sem_g[b]).wait()

    def wb_wait_out(z, b):
      r = row0 + z * _CHUNK

      @pl.when(cid == 0)
      def _():
        pltpu.make_async_copy(rows_v.at[b], out0_hbm.at[pl.ds(r, _CHUNK)],
                              sem_s[b]).wait()

      @pl.when(cid != 0)
      def _():
        pltpu.make_async_copy(rows_v.at[b], out1_hbm.at[pl.ds(r, _CHUNK)],
                              sem_s[b]).wait()

    wb_in(0, 0)
    for z in range(zchunks):
      b = z % 2
      wb_wait_in(z, b)
      wb_out(z, b)
      if z + 1 < zchunks:
        if z >= 1:
          wb_wait_out(z - 1, 1 - b)
        wb_in(z + 1, 1 - b)
    for z in (zchunks - 2, zchunks - 1):
      wb_wait_out(z, z % 2)

  return agg


def _dense(base, p0, p1, w_t, b, n_out):
  D = base.shape[1]
  H = w_t.shape[1]
  BM = 400
  grid = n_out // BM

  def body(base_ref, p0_ref, p1_ref, w_ref, b_ref, o_ref):
    h = base_ref[...] + p0_ref[...] + p1_ref[...]
    acc = jnp.dot(h, w_ref[...], preferred_element_type=jnp.float32)
    acc = acc + b_ref[...]
    o_ref[...] = jnp.where(acc > 0, acc, jnp.exp(acc) - 1.0)

  return pl.pallas_call(
      body,
      grid=(grid,),
      in_specs=[
          pl.BlockSpec((BM, D), lambda i: (i, 0)),
          pl.BlockSpec((BM, D), lambda i: (i, 0)),
          pl.BlockSpec((BM, D), lambda i: (i, 0)),
          pl.BlockSpec((D, H), lambda i: (0, 0)),
          pl.BlockSpec((1, H), lambda i: (0, 0)),
      ],
      out_specs=pl.BlockSpec((BM, H), lambda i: (i, 0)),
      out_shape=jax.ShapeDtypeStruct((n_out, H), jnp.float32),
  )(base, p0, p1, w_t, b.reshape(1, H))


def kernel(features, edge_index, W1, b1, W2, b2):
  N, D = features.shape
  E = edge_index.shape[1]
  H = W1.shape[0]
  NW = _NC * _NS

  N_pad = ((N + 8 + 255) // 256) * 256
  step = NW * _CHUNK * _G
  E_pad = ((E + step - 1) // step) * step

  src, dst = edge_index[0], edge_index[1]
  pad_e = E_pad - E
  if E % NW == 0 and pad_e % NW == 0:
    ppt = pad_e // NW
    cyc = jnp.arange(ppt, dtype=jnp.int32) % (N_pad - N)
    pad_blk = jnp.broadcast_to(cyc, (NW, ppt))
    src_p = jnp.concatenate(
        [src.reshape(NW, E // NW), pad_blk], axis=1).reshape(-1, _CHUNK)
    dst_p = jnp.concatenate(
        [dst.reshape(NW, E // NW), N + pad_blk], axis=1).reshape(-1, _CHUNK)
  else:
    cyc = jnp.arange(pad_e, dtype=jnp.int32) % (N_pad - N)
    src_p = jnp.concatenate([src, cyc]).reshape(-1, _CHUNK)
    dst_p = jnp.concatenate([dst, N + cyc]).reshape(-1, _CHUNK)
  zeros_rows = jnp.zeros((_CHUNK, D), jnp.float32)

  agg1 = _make_agg(N, N_pad, D, E_pad)
  p0, p1 = agg1(features, src_p, dst_p, zeros_rows)
  x = _dense(features, p0, p1, W1.T, b1, N)

  agg2 = _make_agg(N, N_pad, H, E_pad)
  q0, q1 = agg2(x, src_p, dst_p, zeros_rows)
  return _dense(x, q0, q1, W2.T, b2, N)
